# baseline (speedup 1.0000x reference)
import functools

import jax
import jax.numpy as jnp
from jax.experimental import pallas as pl

N, D, E, TOP_K = 8192, 2048, 16, 2
BLOCK = 2048


def _router_body(x_ref, w_ref, b_ref, eps_ref, out_ref, idx_ref):
    x = x_ref[...]
    acc = jnp.dot(x, w_ref[...], preferred_element_type=jnp.float32)
    acc = acc + b_ref[...]
    logits = acc[:, :E]
    nlog = acc[:, E:]
    noisy = logits + eps_ref[...] * jax.nn.softplus(nlog)

    iota = jax.lax.broadcasted_iota(jnp.int32, noisy.shape, 1)
    m1 = jnp.max(noisy, axis=1, keepdims=True)
    i1 = jnp.min(jnp.where(noisy == m1, iota, E), axis=1, keepdims=True)
    masked = jnp.where(iota == i1, -jnp.inf, noisy)
    m2 = jnp.max(masked, axis=1, keepdims=True)
    i2 = jnp.min(jnp.where(masked == m2, iota, E), axis=1, keepdims=True)

    e2 = jnp.exp(m2 - m1)
    inv_denom = 1.0 / (1.0 + e2)
    out = jnp.where(iota == i1, inv_denom,
                    jnp.where(iota == i2, e2 * inv_denom, 0.0))
    out_ref[...] = out
    idx_ref[...] = jnp.concatenate([i1, i2], axis=1)


@functools.partial(jax.jit, static_argnames=("interpret",))
def kernel(x, Wg, bg, Wn, bn, eps, interpret=False):
    grid = (N // BLOCK,)
    out_shapes = (
        jax.ShapeDtypeStruct((N, E), jnp.float32),
        jax.ShapeDtypeStruct((N, TOP_K), jnp.int32),
    )
    sparse, idx = pl.pallas_call(
        _router_body,
        grid=grid,
        in_specs=[
            pl.BlockSpec((BLOCK, D), lambda i: (i, 0)),
            pl.BlockSpec((D, 2 * E), lambda i: (0, 0)),
            pl.BlockSpec((1, 2 * E), lambda i: (0, 0)),
            pl.BlockSpec((BLOCK, E), lambda i: (i, 0)),
        ],
        out_specs=(
            pl.BlockSpec((BLOCK, E), lambda i: (i, 0)),
            pl.BlockSpec((BLOCK, TOP_K), lambda i: (i, 0)),
        ),
        out_shape=out_shapes,
        interpret=interpret,
    )(x, jnp.concatenate([Wg, Wn], axis=1),
      jnp.concatenate([bg, bn]).reshape(1, 2 * E), eps)
    return sparse, idx


# speedup vs baseline: 1.0213x; 1.0213x over previous
import functools

import jax
import jax.numpy as jnp
from jax.experimental import pallas as pl

N, D, E, TOP_K = 8192, 2048, 16, 2
BLOCK = 1024


def _router_body(x_ref, w_ref, b_ref, eps_ref, out_ref, idx_ref):
    x = x_ref[...]
    acc = jnp.dot(x, w_ref[...], preferred_element_type=jnp.float32)
    acc = acc + b_ref[...]
    logits = acc[:, :E]
    nlog = acc[:, E:]
    noisy = logits + eps_ref[...] * jax.nn.softplus(nlog)

    iota = jax.lax.broadcasted_iota(jnp.int32, noisy.shape, 1)
    m1 = jnp.max(noisy, axis=1, keepdims=True)
    i1 = jnp.min(jnp.where(noisy == m1, iota, E), axis=1, keepdims=True)
    masked = jnp.where(iota == i1, -jnp.inf, noisy)
    m2 = jnp.max(masked, axis=1, keepdims=True)
    i2 = jnp.min(jnp.where(masked == m2, iota, E), axis=1, keepdims=True)

    e2 = jnp.exp(m2 - m1)
    inv_denom = 1.0 / (1.0 + e2)
    out = jnp.where(iota == i1, inv_denom,
                    jnp.where(iota == i2, e2 * inv_denom, 0.0))
    out_ref[...] = out
    idx_ref[...] = jnp.concatenate([i1, i2], axis=1)


@functools.partial(jax.jit, static_argnames=("interpret",))
def kernel(x, Wg, bg, Wn, bn, eps, interpret=False):
    grid = (N // BLOCK,)
    out_shapes = (
        jax.ShapeDtypeStruct((N, E), jnp.float32),
        jax.ShapeDtypeStruct((N, TOP_K), jnp.int32),
    )
    sparse, idx = pl.pallas_call(
        _router_body,
        grid=grid,
        in_specs=[
            pl.BlockSpec((BLOCK, D), lambda i: (i, 0)),
            pl.BlockSpec((D, 2 * E), lambda i: (0, 0)),
            pl.BlockSpec((1, 2 * E), lambda i: (0, 0)),
            pl.BlockSpec((BLOCK, E), lambda i: (i, 0)),
        ],
        out_specs=(
            pl.BlockSpec((BLOCK, E), lambda i: (i, 0)),
            pl.BlockSpec((BLOCK, TOP_K), lambda i: (i, 0)),
        ),
        out_shape=out_shapes,
        interpret=interpret,
    )(x, jnp.concatenate([Wg, Wn], axis=1),
      jnp.concatenate([bg, bn]).reshape(1, 2 * E), eps)
    return sparse, idx
